# baseline (device time: 9837 ns/iter reference)
import jax
import jax.numpy as jnp
from jax import lax
from jax.experimental import pallas as pl
from jax.experimental.pallas import tpu as pltpu

N_DEV = 4

_DeviceIdType = getattr(pltpu, "DeviceIdType", None) or pl.DeviceIdType
_CompilerParams = getattr(pltpu, "CompilerParams", None) or pltpu.TPUCompilerParams

_SEND_ORDER = (2, 1, 3)


def kernel(x):
    _, m, n_total = x.shape
    n_per = n_total // N_DEV

    def body(x_hbm, out_ref, xv_ref, send_buf, recv_ref, in_sems, send_sems, recv_sems):
        my_p = lax.axis_index("i")

        in_copies = []
        for k, o in enumerate(_SEND_ORDER + (0,)):
            c = (my_p + o) % N_DEV
            cp = pltpu.make_async_copy(
                x_hbm.at[0, :, pl.ds(c * n_per, n_per)],
                xv_ref.at[k],
                in_sems.at[k],
            )
            cp.start()
            in_copies.append(cp)

        barrier_sem = pltpu.get_barrier_semaphore()
        for o in range(1, N_DEV):
            peer = (my_p + o) % N_DEV
            pl.semaphore_signal(
                barrier_sem,
                inc=1,
                device_id=(peer,),
                device_id_type=_DeviceIdType.MESH,
            )

        for k in range(N_DEV - 1):
            in_copies[k].wait()
            send_buf[k] = xv_ref[k].astype(jnp.bfloat16)

        pl.semaphore_wait(barrier_sem, N_DEV - 1)

        rdmas = []
        for k, o in enumerate(_SEND_ORDER):
            d = (my_p + o) % N_DEV
            rdma = pltpu.make_async_remote_copy(
                src_ref=send_buf.at[k],
                dst_ref=recv_ref.at[k],
                send_sem=send_sems.at[k],
                recv_sem=recv_sems.at[k],
                device_id=(d,),
                device_id_type=_DeviceIdType.MESH,
            )
            rdma.start()
            rdmas.append(rdma)

        in_copies[3].wait()
        acc = xv_ref[3]
        for k in (1, 2, 0):
            rdmas[k].wait_recv()
            acc = acc + recv_ref[k].astype(jnp.float32)
        out_ref[...] = acc.astype(jnp.bfloat16)

        for rdma in rdmas:
            rdma.wait_send()

    return pl.pallas_call(
        body,
        out_shape=jax.ShapeDtypeStruct((m, n_per), jnp.bfloat16),
        in_specs=[pl.BlockSpec(memory_space=pltpu.MemorySpace.HBM)],
        out_specs=pl.BlockSpec(memory_space=pltpu.MemorySpace.VMEM),
        scratch_shapes=[
            pltpu.VMEM((N_DEV, m, n_per), jnp.float32),
            pltpu.VMEM((N_DEV - 1, m, n_per), jnp.bfloat16),
            pltpu.VMEM((N_DEV - 1, m, n_per), jnp.bfloat16),
            pltpu.SemaphoreType.DMA((N_DEV,)),
            pltpu.SemaphoreType.DMA((N_DEV - 1,)),
            pltpu.SemaphoreType.DMA((N_DEV - 1,)),
        ],
        compiler_params=_CompilerParams(collective_id=0),
    )(x)


# device time: 9770 ns/iter; 1.0069x vs baseline; 1.0069x over previous
import jax
import jax.numpy as jnp
from jax import lax
from jax.experimental import pallas as pl
from jax.experimental.pallas import tpu as pltpu

N_DEV = 4

_DeviceIdType = getattr(pltpu, "DeviceIdType", None) or pl.DeviceIdType
_CompilerParams = getattr(pltpu, "CompilerParams", None) or pltpu.TPUCompilerParams

_SEND_ORDER = (2, 1, 3)


def kernel(x):
    _, m, n_total = x.shape
    n_per = n_total // N_DEV
    xb = x.reshape(m, n_total).astype(jnp.bfloat16)

    def body(x_ref, out_ref, recv_ref, send_sems, recv_sems):
        my_p = lax.axis_index("i")

        barrier_sem = pltpu.get_barrier_semaphore()
        for o in _SEND_ORDER:
            peer = (my_p + o) % N_DEV
            pl.semaphore_signal(
                barrier_sem,
                inc=1,
                device_id=(peer,),
                device_id_type=_DeviceIdType.MESH,
            )
        pl.semaphore_wait(barrier_sem, N_DEV - 1)

        rdmas = []
        for k, o in enumerate(_SEND_ORDER):
            d = (my_p + o) % N_DEV
            rdma = pltpu.make_async_remote_copy(
                src_ref=x_ref.at[:, pl.ds(d * n_per, n_per)],
                dst_ref=recv_ref.at[k],
                send_sem=send_sems.at[k],
                recv_sem=recv_sems.at[k],
                device_id=(d,),
                device_id_type=_DeviceIdType.MESH,
            )
            rdma.start()
            rdmas.append(rdma)

        acc = x_ref[:, pl.ds(my_p * n_per, n_per)]
        for k in (1, 2, 0):
            rdmas[k].wait_recv()
            acc = acc + recv_ref[k]
        out_ref[...] = acc

        for rdma in rdmas:
            rdma.wait_send()

    return pl.pallas_call(
        body,
        out_shape=jax.ShapeDtypeStruct((m, n_per), jnp.bfloat16),
        in_specs=[pl.BlockSpec(memory_space=pltpu.MemorySpace.VMEM)],
        out_specs=pl.BlockSpec(memory_space=pltpu.MemorySpace.VMEM),
        scratch_shapes=[
            pltpu.VMEM((N_DEV - 1, m, n_per), jnp.bfloat16),
            pltpu.SemaphoreType.DMA((N_DEV - 1,)),
            pltpu.SemaphoreType.DMA((N_DEV - 1,)),
        ],
        compiler_params=_CompilerParams(collective_id=0),
    )(xb)


# device time: 9668 ns/iter; 1.0175x vs baseline; 1.0106x over previous
import jax
import jax.numpy as jnp
from jax import lax
from jax.experimental import pallas as pl
from jax.experimental.pallas import tpu as pltpu

N_DEV = 4

_DeviceIdType = getattr(pltpu, "DeviceIdType", None) or pl.DeviceIdType
_CompilerParams = getattr(pltpu, "CompilerParams", None) or pltpu.TPUCompilerParams

_SEND_ORDER = (2, 1, 3)


def kernel(x):
    _, m, n_total = x.shape
    n_per = n_total // N_DEV

    def body(x_ref, out_ref, send_buf, recv_ref, send_sems, recv_sems):
        my_p = lax.axis_index("i")

        barrier_sem = pltpu.get_barrier_semaphore()
        for o in _SEND_ORDER:
            peer = (my_p + o) % N_DEV
            pl.semaphore_signal(
                barrier_sem,
                inc=1,
                device_id=(peer,),
                device_id_type=_DeviceIdType.MESH,
            )

        for k, o in enumerate(_SEND_ORDER):
            d = (my_p + o) % N_DEV
            send_buf[k] = x_ref[0, :, pl.ds(d * n_per, n_per)].astype(
                jnp.bfloat16
            )

        pl.semaphore_wait(barrier_sem, N_DEV - 1)

        rdmas = []
        for k, o in enumerate(_SEND_ORDER):
            d = (my_p + o) % N_DEV
            rdma = pltpu.make_async_remote_copy(
                src_ref=send_buf.at[k],
                dst_ref=recv_ref.at[k],
                send_sem=send_sems.at[k],
                recv_sem=recv_sems.at[k],
                device_id=(d,),
                device_id_type=_DeviceIdType.MESH,
            )
            rdma.start()
            rdmas.append(rdma)

        acc = x_ref[0, :, pl.ds(my_p * n_per, n_per)]
        for k in (1, 2, 0):
            rdmas[k].wait_recv()
            acc = acc + recv_ref[k].astype(jnp.float32)
        out_ref[...] = acc.astype(jnp.bfloat16)

        for rdma in rdmas:
            rdma.wait_send()

    return pl.pallas_call(
        body,
        out_shape=jax.ShapeDtypeStruct((m, n_per), jnp.bfloat16),
        in_specs=[pl.BlockSpec(memory_space=pltpu.MemorySpace.VMEM)],
        out_specs=pl.BlockSpec(memory_space=pltpu.MemorySpace.VMEM),
        scratch_shapes=[
            pltpu.VMEM((N_DEV - 1, m, n_per), jnp.bfloat16),
            pltpu.VMEM((N_DEV - 1, m, n_per), jnp.bfloat16),
            pltpu.SemaphoreType.DMA((N_DEV - 1,)),
            pltpu.SemaphoreType.DMA((N_DEV - 1,)),
        ],
        compiler_params=_CompilerParams(collective_id=0),
    )(x)
